# baseline (speedup 1.0000x reference)
import functools

import jax
import jax.numpy as jnp
import numpy as np
from jax import lax
from jax.experimental import pallas as pl
from jax.experimental.pallas import tpu as pltpu
from jax.experimental.pallas import tpu_sc as plsc


def _hilbert_flat(n: int) -> np.ndarray:
    d = np.arange(n * n, dtype=np.int64)
    x = np.zeros_like(d)
    y = np.zeros_like(d)
    t = d.copy()
    s = 1
    while s < n:
        rx = 1 & (t // 2)
        ry = 1 & (t ^ rx)
        swap = ry == 0
        flip = swap & (rx == 1)
        xf = np.where(flip, s - 1 - x, x)
        yf = np.where(flip, s - 1 - y, y)
        xn = np.where(swap, yf, xf)
        yn = np.where(swap, xf, yf)
        x = xn + s * rx
        y = yn + s * ry
        t = t // 4
        s *= 2
    return x * n + y


@functools.cache
def _build(B, H, W, C):
    n_rows = B * H * W
    info = plsc.get_sparse_core_info()
    NW = info.num_cores * info.num_subcores
    NC = info.num_cores
    per_w = n_rows // NW
    CHUNK = 512
    n_chunks = per_w // CHUNK

    mesh = plsc.VectorSubcoreMesh(core_axis_name="c", subcore_axis_name="s")

    @functools.partial(
        pl.kernel,
        mesh=mesh,
        out_type=jax.ShapeDtypeStruct((n_rows, C), jnp.float32),
        compiler_params=pltpu.CompilerParams(use_tc_tiling_on_sc=False),
        scratch_types=[
            pltpu.VMEM((per_w,), jnp.int32),
            pltpu.VMEM((CHUNK, C), jnp.float32),
            pltpu.SemaphoreType.DMA,
        ],
    )
    def gather_kernel(table_hbm, idx_hbm, out_hbm, idx_v, rows, gsem):
        wid = lax.axis_index("s") * NC + lax.axis_index("c")
        base = wid * per_w
        pltpu.sync_copy(idx_hbm.at[pl.ds(base, per_w)], idx_v)

        def body(i, _):
            off = i * CHUNK
            pltpu.async_copy(
                table_hbm.at[idx_v.at[pl.ds(off, CHUNK)]], rows, gsem
            ).wait()
            pltpu.sync_copy(rows, out_hbm.at[pl.ds(base + off, CHUNK)])
            return 0

        lax.fori_loop(0, n_chunks, body, 0)

    flat = _hilbert_flat(H)
    idx_global = (
        np.arange(B, dtype=np.int64)[:, None] * (H * W) + flat[None, :]
    ).reshape(-1).astype(np.int32)
    idx_const = jnp.asarray(idx_global)
    return gather_kernel, idx_const


def kernel(inputs):
    B, H, W, C = inputs.shape
    gather_kernel, idx_const = _build(B, H, W, C)
    table = inputs.reshape(B * H * W, C)
    out = gather_kernel(table, idx_const)
    return out.reshape(B, 1, H * W, C)
